# baseline (speedup 1.0000x reference)
import functools

import jax
import jax.numpy as jnp
from jax import lax
from jax.experimental import pallas as pl
from jax.experimental.pallas import tpu as pltpu
from jax.experimental.pallas import tpu_sc as plsc

B = 16384
FEAT = 512
DG = 64
NW = 32
BPW = B // NW
C = 64
NCH = BPW // C
NB = 32
TB = B // NB

_mesh = plsc.VectorSubcoreMesh(core_axis_name="c", subcore_axis_name="s")


@functools.partial(
    pl.kernel,
    out_type=(
        jax.ShapeDtypeStruct((B, FEAT), jnp.float32),
        jax.ShapeDtypeStruct((B, DG), jnp.float32),
        jax.ShapeDtypeStruct((B, DG), jnp.float32),
        jax.ShapeDtypeStruct((B, DG), jnp.float32),
        jax.ShapeDtypeStruct((B,), jnp.float32),
    ),
    mesh=_mesh,
    scratch_types=(
        pltpu.VMEM((BPW,), jnp.int32),
        pltpu.VMEM((BPW,), jnp.int32),
        pltpu.VMEM((BPW,), jnp.int32),
        pltpu.VMEM((C, FEAT), jnp.float32),
        pltpu.VMEM((C, FEAT), jnp.float32),
        pltpu.VMEM((C, DG), jnp.float32),
        pltpu.VMEM((C, DG), jnp.float32),
        pltpu.VMEM((C, DG), jnp.float32),
        pltpu.VMEM((C, DG), jnp.float32),
        pltpu.VMEM((C, 16), jnp.float32),
        pltpu.VMEM((C, 16), jnp.float32),
        pltpu.VMEM((C,), jnp.int32),
        pltpu.VMEM((C,), jnp.int32),
        pltpu.VMEM((C,), jnp.float32),
        pltpu.SemaphoreType.DMA,
        pltpu.SemaphoreType.DMA,
    ),
    compiler_params=pltpu.CompilerParams(use_tc_tiling_on_sc=False,
                                         needs_layout_passes=False),
)
def _sc_gather(ui_h, pi_h, ni_h, feat_h, gu_h, gi_h, tu_h, beta_h,
               fd_o, gid_o, gu_o, tu_o, bd_o,
               ui_v, pi_v, ni_v, fp, fn, gub, tub, gipb, ginb,
               brp, brn, qp, qn, bdv, sem, sem2):
    cid = lax.axis_index("c")
    sid = lax.axis_index("s")
    wid = sid * 2 + cid
    base = wid * BPW
    pltpu.sync_copy(ui_h.at[pl.ds(base, BPW)], ui_v)
    pltpu.sync_copy(pi_h.at[pl.ds(base, BPW)], pi_v)
    pltpu.sync_copy(ni_h.at[pl.ds(base, BPW)], ni_v)

    def chunk(ci, carry):
        off = ci * C
        uix = ui_v.at[pl.ds(off, C)]
        pix = pi_v.at[pl.ds(off, C)]
        nix = ni_v.at[pl.ds(off, C)]
        for g in range(C // 16):
            s = g * 16
            pv = pi_v[pl.ds(off + s, 16)]
            nv = ni_v[pl.ds(off + s, 16)]
            qp[pl.ds(s, 16)] = lax.shift_right_logical(pv, 4)
            qn[pl.ds(s, 16)] = lax.shift_right_logical(nv, 4)
        cps = [
            pltpu.async_copy(feat_h.at[pix], fp, sem),
            pltpu.async_copy(feat_h.at[nix], fn, sem),
            pltpu.async_copy(gu_h.at[uix], gub, sem),
            pltpu.async_copy(tu_h.at[uix], tub, sem),
            pltpu.async_copy(gi_h.at[pix], gipb, sem),
            pltpu.async_copy(gi_h.at[nix], ginb, sem),
            pltpu.async_copy(beta_h.at[qp], brp, sem),
            pltpu.async_copy(beta_h.at[qn], brn, sem),
        ]
        for cp in cps:
            cp.wait()

        lanes = lax.iota(jnp.int32, 16)
        for g in range(C // 16):
            s = g * 16
            pv = pi_v[pl.ds(off + s, 16)]
            nv = ni_v[pl.ds(off + s, 16)]
            rows = lanes + s
            valp = plsc.load_gather(brp, [rows, pv & 15])
            valn = plsc.load_gather(brn, [rows, nv & 15])
            bdv[pl.ds(s, 16)] = valp - valn

        def frow(r, c2):
            for k in range(FEAT // 16):
                s = k * 16
                fp[r, pl.ds(s, 16)] = fp[r, pl.ds(s, 16)] - fn[r, pl.ds(s, 16)]
            for k in range(DG // 16):
                s = k * 16
                gipb[r, pl.ds(s, 16)] = (
                    gipb[r, pl.ds(s, 16)] - ginb[r, pl.ds(s, 16)]
                )
            return c2

        lax.fori_loop(0, C, frow, 0)

        dst = pl.ds(base + off, C)
        sts = [
            pltpu.async_copy(fp, fd_o.at[dst], sem2),
            pltpu.async_copy(gipb, gid_o.at[dst], sem2),
            pltpu.async_copy(gub, gu_o.at[dst], sem2),
            pltpu.async_copy(tub, tu_o.at[dst], sem2),
            pltpu.async_copy(bdv, bd_o.at[dst], sem2),
        ]
        for cp in sts:
            cp.wait()
        return carry

    lax.fori_loop(0, NCH, chunk, 0)


def _tc_body(fd_ref, gid_ref, gu_ref, tu_ref, bd_ref, e_ref, vb_ref,
             o_ref):
    fd = fd_ref[...]
    p = jnp.dot(fd, e_ref[...], preferred_element_type=jnp.float32)
    vis = jnp.sum(tu_ref[...] * p, axis=1)
    lat = jnp.sum(gu_ref[...] * gid_ref[...], axis=1)
    vbt = jnp.sum(fd * vb_ref[...], axis=1)
    o_ref[...] = bd_ref[...] + (vis + lat + vbt)[None, None, :]


_tc_dense = pl.pallas_call(
    _tc_body,
    grid=(NB,),
    in_specs=[
        pl.BlockSpec((TB, FEAT), lambda b: (b, 0)),
        pl.BlockSpec((TB, DG), lambda b: (b, 0)),
        pl.BlockSpec((TB, DG), lambda b: (b, 0)),
        pl.BlockSpec((TB, DG), lambda b: (b, 0)),
        pl.BlockSpec((1, 1, TB), lambda b: (b, 0, 0)),
        pl.BlockSpec((FEAT, DG), lambda b: (0, 0)),
        pl.BlockSpec((1, FEAT), lambda b: (0, 0)),
    ],
    out_specs=pl.BlockSpec((1, 1, TB), lambda b: (b, 0, 0)),
    out_shape=jax.ShapeDtypeStruct((NB, 1, TB), jnp.float32),
)


def kernel(ui, pi, ni, features, gamma_users, gamma_items, theta_users,
           embedding, beta_items, visual_bias):
    ui = ui.astype(jnp.int32)
    pi = pi.astype(jnp.int32)
    ni = ni.astype(jnp.int32)
    beta16 = beta_items.reshape(100000 // 16, 16)
    fd, gid, gug, tug, bd = _sc_gather(
        ui, pi, ni, features, gamma_users, gamma_items, theta_users,
        beta16)
    out = _tc_dense(fd, gid, gug, tug, bd.reshape(NB, 1, TB),
                    embedding, visual_bias.reshape(1, FEAT))
    return out.reshape(B)


# speedup vs baseline: 1.6384x; 1.6384x over previous
import functools

import jax
import jax.numpy as jnp
from jax import lax
from jax.experimental import pallas as pl
from jax.experimental.pallas import tpu as pltpu
from jax.experimental.pallas import tpu_sc as plsc

B = 16384
FEAT = 512
DG = 64
W128 = 128
NW = 32
BPW = B // NW
C = 64
NCH = BPW // C
NB = 32
TB = B // NB
L = 16

_mesh = plsc.VectorSubcoreMesh(core_axis_name="c", subcore_axis_name="s")


@functools.partial(
    pl.kernel,
    out_type=(
        jax.ShapeDtypeStruct((B, FEAT), jnp.float32),
        jax.ShapeDtypeStruct((B, W128), jnp.float32),
        jax.ShapeDtypeStruct((B, W128), jnp.float32),
        jax.ShapeDtypeStruct((B, W128), jnp.float32),
        jax.ShapeDtypeStruct((B, W128), jnp.float32),
        jax.ShapeDtypeStruct((B,), jnp.float32),
        jax.ShapeDtypeStruct((B,), jnp.float32),
        jax.ShapeDtypeStruct((B,), jnp.float32),
        jax.ShapeDtypeStruct((B,), jnp.float32),
    ),
    mesh=_mesh,
    scratch_types=(
        pltpu.VMEM((BPW,), jnp.int32),
        pltpu.VMEM((BPW,), jnp.int32),
        pltpu.VMEM((BPW,), jnp.int32),
        pltpu.VMEM((BPW,), jnp.int32),
        pltpu.VMEM((BPW,), jnp.int32),
        pltpu.VMEM((BPW,), jnp.int32),
        pltpu.VMEM((BPW,), jnp.float32),
        pltpu.VMEM((BPW,), jnp.float32),
        pltpu.VMEM((BPW,), jnp.float32),
        pltpu.VMEM((BPW,), jnp.float32),
        pltpu.VMEM((BPW,), jnp.float32),
        pltpu.VMEM((C, FEAT), jnp.float32),
        pltpu.VMEM((C, FEAT), jnp.float32),
        pltpu.VMEM((C, W128), jnp.float32),
        pltpu.VMEM((C, W128), jnp.float32),
        pltpu.VMEM((C, W128), jnp.float32),
        pltpu.VMEM((C, W128), jnp.float32),
        pltpu.SemaphoreType.DMA,
        pltpu.SemaphoreType.DMA,
    ),
)
def _sc_gather(ui_h, pi_h, ni_h, feat_h, gu_h, gi_h, tu_h, beta_h,
               fd_o, gu_o, tu_o, gp_o, gn_o, bd_o, pu_o, pp_o, pn_o,
               ui_v, pi_v, ni_v, qu, qp, qn, bpv, bnv, puv, ppv, pnv,
               fp, fn, gub, tub, gpb, gnb, sem, sem2):
    cid = lax.axis_index("c")
    sid = lax.axis_index("s")
    wid = sid * 2 + cid
    base = wid * BPW
    pltpu.sync_copy(ui_h.at[pl.ds(base, BPW)], ui_v)
    pltpu.sync_copy(pi_h.at[pl.ds(base, BPW)], pi_v)
    pltpu.sync_copy(ni_h.at[pl.ds(base, BPW)], ni_v)

    def prep(g, carry):
        s = g * L
        uv = ui_v[pl.ds(s, L)]
        pv = pi_v[pl.ds(s, L)]
        nv = ni_v[pl.ds(s, L)]
        qu[pl.ds(s, L)] = lax.shift_right_logical(uv, 1)
        qp[pl.ds(s, L)] = lax.shift_right_logical(pv, 1)
        qn[pl.ds(s, L)] = lax.shift_right_logical(nv, 1)
        puv[pl.ds(s, L)] = (uv & 1).astype(jnp.float32)
        ppv[pl.ds(s, L)] = (pv & 1).astype(jnp.float32)
        pnv[pl.ds(s, L)] = (nv & 1).astype(jnp.float32)
        return carry

    lax.fori_loop(0, BPW // L, prep, 0, unroll=4)

    bcp = pltpu.async_copy(beta_h.at[pi_v], bpv, sem)
    bcn = pltpu.async_copy(beta_h.at[ni_v], bnv, sem)
    bcp.wait()
    bcn.wait()

    def bdiff(g, carry):
        s = g * L
        bpv[pl.ds(s, L)] = bpv[pl.ds(s, L)] - bnv[pl.ds(s, L)]
        return carry

    lax.fori_loop(0, BPW // L, bdiff, 0, unroll=4)

    st0 = [
        pltpu.async_copy(bpv, bd_o.at[pl.ds(base, BPW)], sem2),
        pltpu.async_copy(puv, pu_o.at[pl.ds(base, BPW)], sem2),
        pltpu.async_copy(ppv, pp_o.at[pl.ds(base, BPW)], sem2),
        pltpu.async_copy(pnv, pn_o.at[pl.ds(base, BPW)], sem2),
    ]

    def chunk(ci, carry):
        off = ci * C
        cps = [
            pltpu.async_copy(feat_h.at[pi_v.at[pl.ds(off, C)]], fp, sem),
            pltpu.async_copy(feat_h.at[ni_v.at[pl.ds(off, C)]], fn, sem),
            pltpu.async_copy(gu_h.at[qu.at[pl.ds(off, C)]], gub, sem),
            pltpu.async_copy(tu_h.at[qu.at[pl.ds(off, C)]], tub, sem),
            pltpu.async_copy(gi_h.at[qp.at[pl.ds(off, C)]], gpb, sem),
            pltpu.async_copy(gi_h.at[qn.at[pl.ds(off, C)]], gnb, sem),
        ]
        for cp in cps:
            cp.wait()

        def frow(r, c2):
            for k in range(FEAT // L):
                s = k * L
                fp[r, pl.ds(s, L)] = fp[r, pl.ds(s, L)] - fn[r, pl.ds(s, L)]
            return c2

        lax.fori_loop(0, C, frow, 0)

        dst = pl.ds(base + off, C)
        sts = [
            pltpu.async_copy(fp, fd_o.at[dst], sem2),
            pltpu.async_copy(gub, gu_o.at[dst], sem2),
            pltpu.async_copy(tub, tu_o.at[dst], sem2),
            pltpu.async_copy(gpb, gp_o.at[dst], sem2),
            pltpu.async_copy(gnb, gn_o.at[dst], sem2),
        ]
        for cp in sts:
            cp.wait()
        return carry

    lax.fori_loop(0, NCH, chunk, 0)
    for cp in st0:
        cp.wait()


def _half(pair, par_col):
    sel = par_col > 0.5
    return jnp.where(sel, pair[:, DG:], pair[:, :DG])


def _tc_body(fd_ref, gu_ref, tu_ref, gp_ref, gn_ref, bd_ref, pu_ref, pp_ref,
             pn_ref, e_ref, vb_ref, o_ref):
    fd = fd_ref[...]
    pu_col = pu_ref[0, 0, :][:, None]
    pp_col = pp_ref[0, 0, :][:, None]
    pn_col = pn_ref[0, 0, :][:, None]
    gu = _half(gu_ref[...], pu_col)
    tu = _half(tu_ref[...], pu_col)
    gid = _half(gp_ref[...], pp_col) - _half(gn_ref[...], pn_col)
    p = jnp.dot(fd, e_ref[...], preferred_element_type=jnp.float32)
    vis = jnp.sum(tu * p, axis=1)
    lat = jnp.sum(gu * gid, axis=1)
    vbt = jnp.sum(fd * vb_ref[...], axis=1)
    o_ref[...] = bd_ref[...] + (vis + lat + vbt)[None, None, :]


_tc_dense = pl.pallas_call(
    _tc_body,
    grid=(NB,),
    in_specs=[
        pl.BlockSpec((TB, FEAT), lambda b: (b, 0)),
        pl.BlockSpec((TB, W128), lambda b: (b, 0)),
        pl.BlockSpec((TB, W128), lambda b: (b, 0)),
        pl.BlockSpec((TB, W128), lambda b: (b, 0)),
        pl.BlockSpec((TB, W128), lambda b: (b, 0)),
        pl.BlockSpec((1, 1, TB), lambda b: (b, 0, 0)),
        pl.BlockSpec((1, 1, TB), lambda b: (b, 0, 0)),
        pl.BlockSpec((1, 1, TB), lambda b: (b, 0, 0)),
        pl.BlockSpec((1, 1, TB), lambda b: (b, 0, 0)),
        pl.BlockSpec((FEAT, DG), lambda b: (0, 0)),
        pl.BlockSpec((1, FEAT), lambda b: (0, 0)),
    ],
    out_specs=pl.BlockSpec((1, 1, TB), lambda b: (b, 0, 0)),
    out_shape=jax.ShapeDtypeStruct((NB, 1, TB), jnp.float32),
)


def kernel(ui, pi, ni, features, gamma_users, gamma_items, theta_users,
           embedding, beta_items, visual_bias):
    ui = ui.astype(jnp.int32)
    pi = pi.astype(jnp.int32)
    ni = ni.astype(jnp.int32)
    gu2 = gamma_users.reshape(-1, W128)
    gi2 = gamma_items.reshape(-1, W128)
    tu2 = theta_users.reshape(-1, W128)
    beta_flat = beta_items.reshape(-1)
    fd, gu128, tu128, gp128, gn128, bd, pu, pp, pn = _sc_gather(
        ui, pi, ni, features, gu2, gi2, tu2, beta_flat)
    out = _tc_dense(fd, gu128, tu128, gp128, gn128,
                    bd.reshape(NB, 1, TB), pu.reshape(NB, 1, TB),
                    pp.reshape(NB, 1, TB), pn.reshape(NB, 1, TB),
                    embedding, visual_bias.reshape(1, FEAT))
    return out.reshape(B)


# speedup vs baseline: 1.7484x; 1.0671x over previous
import functools

import jax
import jax.numpy as jnp
from jax import lax
from jax.experimental import pallas as pl
from jax.experimental.pallas import tpu as pltpu
from jax.experimental.pallas import tpu_sc as plsc

B = 16384
FEAT = 512
DG = 64
W128 = 128
NW = 32
BPW = B // NW
CF = 32
NCF = BPW // CF
CN = 64
NCN = BPW // CN
NB = 32
TB = B // NB
L = 16

_mesh = plsc.VectorSubcoreMesh(core_axis_name="c", subcore_axis_name="s")


def _wid_base():
    return (lax.axis_index("s") * 2 + lax.axis_index("c")) * BPW


@functools.partial(
    pl.kernel,
    out_type=(
        jax.ShapeDtypeStruct((B, FEAT), jnp.float32),
        jax.ShapeDtypeStruct((B,), jnp.float32),
    ),
    mesh=_mesh,
    scratch_types=(
        pltpu.VMEM((BPW,), jnp.int32),
        pltpu.VMEM((BPW,), jnp.int32),
        pltpu.VMEM((BPW,), jnp.float32),
        pltpu.VMEM((BPW,), jnp.float32),
        pltpu.VMEM((CF, FEAT), jnp.float32),
        pltpu.VMEM((CF, FEAT), jnp.float32),
        pltpu.VMEM((CF, FEAT), jnp.float32),
        pltpu.VMEM((CF, FEAT), jnp.float32),
        pltpu.SemaphoreType.DMA,
        pltpu.SemaphoreType.DMA,
        pltpu.SemaphoreType.DMA,
        pltpu.SemaphoreType.DMA,
        pltpu.SemaphoreType.DMA,
    ),
)
def _sc_feat(pi_h, ni_h, feat_h, beta_h, fd_o, bd_o,
             pi_v, ni_v, bpv, bnv, fp0, fn0, fp1, fn1,
             semb, semg0, semg1, sems0, sems1):
    base = _wid_base()
    pltpu.sync_copy(pi_h.at[pl.ds(base, BPW)], pi_v)
    pltpu.sync_copy(ni_h.at[pl.ds(base, BPW)], ni_v)

    bcp = pltpu.async_copy(beta_h.at[pi_v], bpv, semb)
    bcn = pltpu.async_copy(beta_h.at[ni_v], bnv, semb)

    fps = (fp0, fp1)
    fns = (fn0, fn1)
    semg = (semg0, semg1)
    sems = (sems0, sems1)

    def fire(ci):
        s = ci % 2
        return [
            pltpu.async_copy(feat_h.at[pi_v.at[pl.ds(ci * CF, CF)]],
                             fps[s], semg[s]),
            pltpu.async_copy(feat_h.at[ni_v.at[pl.ds(ci * CF, CF)]],
                             fns[s], semg[s]),
        ]

    def process(ci, gcp):
        s = ci % 2
        for cp in gcp:
            cp.wait()
        fp, fn = fps[s], fns[s]

        def frow(r, c2):
            for k in range(FEAT // L):
                o = k * L
                fp[r, pl.ds(o, L)] = fp[r, pl.ds(o, L)] - fn[r, pl.ds(o, L)]
            return c2

        lax.fori_loop(0, CF, frow, 0)
        return pltpu.async_copy(fp, fd_o.at[pl.ds(base + ci * CF, CF)],
                                sems[s])

    g = {}
    st = {}
    for ci in range(NCF):
        if ci >= 2:
            st[ci - 2].wait()
        g[ci] = fire(ci)
        if ci >= 1:
            st[ci - 1] = process(ci - 1, g[ci - 1])
    st[NCF - 1] = process(NCF - 1, g[NCF - 1])
    st[NCF - 2].wait()
    st[NCF - 1].wait()

    bcp.wait()
    bcn.wait()

    def bdiff(gi, carry):
        s = gi * L
        bpv[pl.ds(s, L)] = bpv[pl.ds(s, L)] - bnv[pl.ds(s, L)]
        return carry

    lax.fori_loop(0, BPW // L, bdiff, 0, unroll=4)
    pltpu.sync_copy(bpv, bd_o.at[pl.ds(base, BPW)])


@functools.partial(
    pl.kernel,
    out_type=(
        jax.ShapeDtypeStruct((B, W128), jnp.float32),
        jax.ShapeDtypeStruct((B, W128), jnp.float32),
    ),
    mesh=_mesh,
    scratch_types=(
        pltpu.VMEM((BPW + L,), jnp.int32),
        pltpu.VMEM((BPW + L,), jnp.int32),
        pltpu.VMEM((BPW + L,), jnp.int32),
        pltpu.VMEM((BPW,), jnp.int32),
        pltpu.VMEM((BPW,), jnp.int32),
        pltpu.VMEM((BPW,), jnp.int32),
        pltpu.VMEM((CN, W128), jnp.float32),
        pltpu.VMEM((CN, W128), jnp.float32),
        pltpu.VMEM((CN, W128), jnp.float32),
        pltpu.VMEM((CN, W128), jnp.float32),
        pltpu.VMEM((CN, W128), jnp.float32),
        pltpu.VMEM((CN, W128), jnp.float32),
        pltpu.VMEM((CN, W128), jnp.float32),
        pltpu.VMEM((CN, W128), jnp.float32),
        pltpu.SemaphoreType.DMA,
        pltpu.SemaphoreType.DMA,
        pltpu.SemaphoreType.DMA,
        pltpu.SemaphoreType.DMA,
    ),
)
def _sc_narrow(ui_h, pi_h, ni_h, gu_h, gi_h, tu_h, ugut_o, gid_o,
               ui_v, pi_v, ni_v, qu, qp, qn,
               gub0, tub0, gpb0, gnb0, gub1, tub1, gpb1, gnb1,
               semg0, semg1, sems0, sems1):
    base = _wid_base()
    pltpu.sync_copy(ui_h.at[pl.ds(base, BPW)], ui_v.at[pl.ds(0, BPW)])
    pltpu.sync_copy(pi_h.at[pl.ds(base, BPW)], pi_v.at[pl.ds(0, BPW)])
    pltpu.sync_copy(ni_h.at[pl.ds(base, BPW)], ni_v.at[pl.ds(0, BPW)])

    def prep(g, carry):
        s = g * L
        qu[pl.ds(s, L)] = lax.shift_right_logical(ui_v[pl.ds(s, L)], 1)
        qp[pl.ds(s, L)] = lax.shift_right_logical(pi_v[pl.ds(s, L)], 1)
        qn[pl.ds(s, L)] = lax.shift_right_logical(ni_v[pl.ds(s, L)], 1)
        return carry

    lax.fori_loop(0, BPW // L, prep, 0, unroll=4)

    gubs = (gub0, gub1)
    tubs = (tub0, tub1)
    gpbs = (gpb0, gpb1)
    gnbs = (gnb0, gnb1)
    semg = (semg0, semg1)
    sems = (sems0, sems1)

    def fire(ci):
        s = ci % 2
        ix = pl.ds(ci * CN, CN)
        return [
            pltpu.async_copy(gu_h.at[qu.at[ix]], gubs[s], semg[s]),
            pltpu.async_copy(tu_h.at[qu.at[ix]], tubs[s], semg[s]),
            pltpu.async_copy(gi_h.at[qp.at[ix]], gpbs[s], semg[s]),
            pltpu.async_copy(gi_h.at[qn.at[ix]], gnbs[s], semg[s]),
        ]

    def process(ci, gcp):
        s = ci % 2
        for cp in gcp:
            cp.wait()
        gub, tub, gpb, gnb = gubs[s], tubs[s], gpbs[s], gnbs[s]
        off = ci * CN

        def row(r, c2):
            su = (ui_v[pl.ds(off + r, L)][0] & 1) * DG
            sp = (pi_v[pl.ds(off + r, L)][0] & 1) * DG
            sn = (ni_v[pl.ds(off + r, L)][0] & 1) * DG
            for k in range(DG // L):
                o = k * L
                gub[r, pl.ds(o, L)] = gub[r, pl.ds(su + o, L)]
                gub[r, pl.ds(DG + o, L)] = tub[r, pl.ds(su + o, L)]
                gpb[r, pl.ds(o, L)] = (gpb[r, pl.ds(sp + o, L)]
                                       - gnb[r, pl.ds(sn + o, L)])
            return c2

        lax.fori_loop(0, CN, row, 0)
        dst = pl.ds(base + off, CN)
        return [
            pltpu.async_copy(gub, ugut_o.at[dst], sems[s]),
            pltpu.async_copy(gpb, gid_o.at[dst], sems[s]),
        ]

    g = {}
    st = {}
    for ci in range(NCN):
        if ci >= 2:
            for cp in st[ci - 2]:
                cp.wait()
        g[ci] = fire(ci)
        if ci >= 1:
            st[ci - 1] = process(ci - 1, g[ci - 1])
    st[NCN - 1] = process(NCN - 1, g[NCN - 1])
    for ci in (NCN - 2, NCN - 1):
        for cp in st[ci]:
            cp.wait()


def _tc_body(fd_ref, ugut_ref, gid_ref, bd_ref, e_ref, vb_ref, o_ref):
    fd = fd_ref[...]
    ugut = ugut_ref[...]
    gu = ugut[:, :DG]
    tu = ugut[:, DG:]
    gid = gid_ref[:, :DG]
    p = jnp.dot(fd, e_ref[...], preferred_element_type=jnp.float32)
    vis = jnp.sum(tu * p, axis=1)
    lat = jnp.sum(gu * gid, axis=1)
    vbt = jnp.sum(fd * vb_ref[...], axis=1)
    o_ref[...] = bd_ref[...] + (vis + lat + vbt)[None, None, :]


_tc_dense = pl.pallas_call(
    _tc_body,
    grid=(NB,),
    in_specs=[
        pl.BlockSpec((TB, FEAT), lambda b: (b, 0)),
        pl.BlockSpec((TB, W128), lambda b: (b, 0)),
        pl.BlockSpec((TB, W128), lambda b: (b, 0)),
        pl.BlockSpec((1, 1, TB), lambda b: (b, 0, 0)),
        pl.BlockSpec((FEAT, DG), lambda b: (0, 0)),
        pl.BlockSpec((1, FEAT), lambda b: (0, 0)),
    ],
    out_specs=pl.BlockSpec((1, 1, TB), lambda b: (b, 0, 0)),
    out_shape=jax.ShapeDtypeStruct((NB, 1, TB), jnp.float32),
)


def kernel(ui, pi, ni, features, gamma_users, gamma_items, theta_users,
           embedding, beta_items, visual_bias):
    ui = ui.astype(jnp.int32)
    pi = pi.astype(jnp.int32)
    ni = ni.astype(jnp.int32)
    gu2 = gamma_users.reshape(-1, W128)
    gi2 = gamma_items.reshape(-1, W128)
    tu2 = theta_users.reshape(-1, W128)
    beta_flat = beta_items.reshape(-1)
    fd, bd = _sc_feat(pi, ni, features, beta_flat)
    ugut, gid128 = _sc_narrow(ui, pi, ni, gu2, gi2, tu2)
    out = _tc_dense(fd, ugut, gid128, bd.reshape(NB, 1, TB),
                    embedding, visual_bias.reshape(1, FEAT))
    return out.reshape(B)


# speedup vs baseline: 1.8586x; 1.0630x over previous
import functools

import jax
import jax.numpy as jnp
from jax import lax
from jax.experimental import pallas as pl
from jax.experimental.pallas import tpu as pltpu
from jax.experimental.pallas import tpu_sc as plsc

B = 16384
FEAT = 512
DG = 64
W128 = 128
NW = 32
BPW = B // NW
CF = 32
NCF = BPW // CF
CN = 64
NCN = BPW // CN
NB = 32
TB = B // NB
L = 16

_mesh = plsc.VectorSubcoreMesh(core_axis_name="c", subcore_axis_name="s")


def _wid_base():
    return (lax.axis_index("s") * 2 + lax.axis_index("c")) * BPW


@functools.partial(
    pl.kernel,
    out_type=(
        jax.ShapeDtypeStruct((B, FEAT), jnp.float32),
        jax.ShapeDtypeStruct((B,), jnp.float32),
    ),
    mesh=_mesh,
    scratch_types=(
        pltpu.VMEM((BPW,), jnp.int32),
        pltpu.VMEM((BPW,), jnp.int32),
        pltpu.VMEM((BPW,), jnp.float32),
        pltpu.VMEM((BPW,), jnp.float32),
        pltpu.VMEM((CF, FEAT), jnp.float32),
        pltpu.VMEM((CF, FEAT), jnp.float32),
        pltpu.VMEM((CF, FEAT), jnp.float32),
        pltpu.VMEM((CF, FEAT), jnp.float32),
        pltpu.SemaphoreType.DMA,
        pltpu.SemaphoreType.DMA,
        pltpu.SemaphoreType.DMA,
        pltpu.SemaphoreType.DMA,
        pltpu.SemaphoreType.DMA,
    ),
)
def _sc_feat(pi_h, ni_h, feat_h, beta_h, fd_o, bd_o,
             pi_v, ni_v, bpv, bnv, fp0, fn0, fp1, fn1,
             semb, semg0, semg1, sems0, sems1):
    base = _wid_base()
    pltpu.sync_copy(pi_h.at[pl.ds(base, BPW)], pi_v)
    pltpu.sync_copy(ni_h.at[pl.ds(base, BPW)], ni_v)

    bcp = pltpu.async_copy(beta_h.at[pi_v], bpv, semb)
    bcn = pltpu.async_copy(beta_h.at[ni_v], bnv, semb)

    fps = (fp0, fp1)
    fns = (fn0, fn1)
    semg = (semg0, semg1)
    sems = (sems0, sems1)

    def fire(ci):
        s = ci % 2
        return [
            pltpu.async_copy(feat_h.at[pi_v.at[pl.ds(ci * CF, CF)]],
                             fps[s], semg[s]),
            pltpu.async_copy(feat_h.at[ni_v.at[pl.ds(ci * CF, CF)]],
                             fns[s], semg[s]),
        ]

    def process(ci, gcp):
        s = ci % 2
        for cp in gcp:
            cp.wait()
        fp, fn = fps[s], fns[s]

        def frow(r, c2):
            for k in range(FEAT // L):
                o = k * L
                fp[r, pl.ds(o, L)] = fp[r, pl.ds(o, L)] - fn[r, pl.ds(o, L)]
            return c2

        lax.fori_loop(0, CF, frow, 0)
        return pltpu.async_copy(fp, fd_o.at[pl.ds(base + ci * CF, CF)],
                                sems[s])

    g = {}
    st = {}
    for ci in range(NCF):
        if ci >= 2:
            st[ci - 2].wait()
        g[ci] = fire(ci)
        if ci >= 1:
            st[ci - 1] = process(ci - 1, g[ci - 1])
    st[NCF - 1] = process(NCF - 1, g[NCF - 1])
    st[NCF - 2].wait()
    st[NCF - 1].wait()

    bcp.wait()
    bcn.wait()

    def bdiff(gi, carry):
        s = gi * L
        bpv[pl.ds(s, L)] = bpv[pl.ds(s, L)] - bnv[pl.ds(s, L)]
        return carry

    lax.fori_loop(0, BPW // L, bdiff, 0, unroll=4)
    pltpu.sync_copy(bpv, bd_o.at[pl.ds(base, BPW)])


@functools.partial(
    pl.kernel,
    out_type=(
        jax.ShapeDtypeStruct((B, W128), jnp.float32),
        jax.ShapeDtypeStruct((B, W128), jnp.float32),
    ),
    mesh=_mesh,
    scratch_types=(
        pltpu.VMEM((BPW + L,), jnp.int32),
        pltpu.VMEM((BPW + L,), jnp.int32),
        pltpu.VMEM((BPW + L,), jnp.int32),
        pltpu.VMEM((BPW,), jnp.int32),
        pltpu.VMEM((BPW,), jnp.int32),
        pltpu.VMEM((BPW,), jnp.int32),
        pltpu.VMEM((CN, W128), jnp.float32),
        pltpu.VMEM((CN, W128), jnp.float32),
        pltpu.VMEM((CN, W128), jnp.float32),
        pltpu.VMEM((CN, W128), jnp.float32),
        pltpu.VMEM((CN, W128), jnp.float32),
        pltpu.VMEM((CN, W128), jnp.float32),
        pltpu.VMEM((CN, W128), jnp.float32),
        pltpu.VMEM((CN, W128), jnp.float32),
        pltpu.SemaphoreType.DMA,
        pltpu.SemaphoreType.DMA,
        pltpu.SemaphoreType.DMA,
        pltpu.SemaphoreType.DMA,
    ),
)
def _sc_narrow(ui_h, pi_h, ni_h, gu_h, gi_h, tu_h, ugut_o, gid_o,
               ui_v, pi_v, ni_v, qu, qp, qn,
               gub0, tub0, gpb0, gnb0, gub1, tub1, gpb1, gnb1,
               semg0, semg1, sems0, sems1):
    base = _wid_base()
    pltpu.sync_copy(ui_h.at[pl.ds(base, BPW)], ui_v.at[pl.ds(0, BPW)])
    pltpu.sync_copy(pi_h.at[pl.ds(base, BPW)], pi_v.at[pl.ds(0, BPW)])
    pltpu.sync_copy(ni_h.at[pl.ds(base, BPW)], ni_v.at[pl.ds(0, BPW)])

    def prep(g, carry):
        s = g * L
        qu[pl.ds(s, L)] = lax.shift_right_logical(ui_v[pl.ds(s, L)], 1)
        qp[pl.ds(s, L)] = lax.shift_right_logical(pi_v[pl.ds(s, L)], 1)
        qn[pl.ds(s, L)] = lax.shift_right_logical(ni_v[pl.ds(s, L)], 1)
        return carry

    lax.fori_loop(0, BPW // L, prep, 0, unroll=4)

    gubs = (gub0, gub1)
    tubs = (tub0, tub1)
    gpbs = (gpb0, gpb1)
    gnbs = (gnb0, gnb1)
    semg = (semg0, semg1)
    sems = (sems0, sems1)

    def fire(ci):
        s = ci % 2
        ix = pl.ds(ci * CN, CN)
        return [
            pltpu.async_copy(gu_h.at[qu.at[ix]], gubs[s], semg[s]),
            pltpu.async_copy(tu_h.at[qu.at[ix]], tubs[s], semg[s]),
            pltpu.async_copy(gi_h.at[qp.at[ix]], gpbs[s], semg[s]),
            pltpu.async_copy(gi_h.at[qn.at[ix]], gnbs[s], semg[s]),
        ]

    def process(ci, gcp):
        s = ci % 2
        for cp in gcp:
            cp.wait()
        gub, tub, gpb, gnb = gubs[s], tubs[s], gpbs[s], gnbs[s]
        off = ci * CN

        def row(r, c2):
            su = (ui_v[pl.ds(off + r, L)][0] & 1) * DG
            sp = (pi_v[pl.ds(off + r, L)][0] & 1) * DG
            sn = (ni_v[pl.ds(off + r, L)][0] & 1) * DG
            for k in range(DG // L):
                o = k * L
                gub[r, pl.ds(o, L)] = gub[r, pl.ds(su + o, L)]
                gub[r, pl.ds(DG + o, L)] = tub[r, pl.ds(su + o, L)]
                gpb[r, pl.ds(o, L)] = (gpb[r, pl.ds(sp + o, L)]
                                       - gnb[r, pl.ds(sn + o, L)])
            return c2

        lax.fori_loop(0, CN, row, 0)
        dst = pl.ds(base + off, CN)
        return [
            pltpu.async_copy(gub, ugut_o.at[dst], sems[s]),
            pltpu.async_copy(gpb, gid_o.at[dst], sems[s]),
        ]

    g = {}
    st = {}
    for ci in range(NCN):
        if ci >= 2:
            for cp in st[ci - 2]:
                cp.wait()
        g[ci] = fire(ci)
        if ci >= 1:
            st[ci - 1] = process(ci - 1, g[ci - 1])
    st[NCN - 1] = process(NCN - 1, g[NCN - 1])
    for ci in (NCN - 2, NCN - 1):
        for cp in st[ci]:
            cp.wait()


def _tc_mm_body(fd_ref, e_ref, p_ref):
    p_ref[...] = jnp.dot(fd_ref[...], e_ref[...],
                         preferred_element_type=jnp.float32)


_tc_mm = pl.pallas_call(
    _tc_mm_body,
    grid=(NB,),
    in_specs=[
        pl.BlockSpec((TB, FEAT), lambda b: (b, 0)),
        pl.BlockSpec((FEAT, W128), lambda b: (0, 0)),
    ],
    out_specs=pl.BlockSpec((TB, W128), lambda b: (b, 0)),
    out_shape=jax.ShapeDtypeStruct((B, W128), jnp.float32),
)


def _tc_fin_body(ugut_ref, gid_ref, p_ref, bd_ref, o_ref):
    p = p_ref[...]
    ugut = ugut_ref[...]
    acc = (ugut[:, :DG] * gid_ref[:, :DG]
           + ugut[:, DG:] * p[:, :DG])
    o_ref[...] = (bd_ref[...]
                  + (jnp.sum(acc, axis=1) + p[:, DG])[None, None, :])


_tc_fin = pl.pallas_call(
    _tc_fin_body,
    grid=(NB,),
    in_specs=[
        pl.BlockSpec((TB, W128), lambda b: (b, 0)),
        pl.BlockSpec((TB, W128), lambda b: (b, 0)),
        pl.BlockSpec((TB, W128), lambda b: (b, 0)),
        pl.BlockSpec((1, 1, TB), lambda b: (b, 0, 0)),
    ],
    out_specs=pl.BlockSpec((1, 1, TB), lambda b: (b, 0, 0)),
    out_shape=jax.ShapeDtypeStruct((NB, 1, TB), jnp.float32),
)


def kernel(ui, pi, ni, features, gamma_users, gamma_items, theta_users,
           embedding, beta_items, visual_bias):
    ui = ui.astype(jnp.int32)
    pi = pi.astype(jnp.int32)
    ni = ni.astype(jnp.int32)
    gu2 = gamma_users.reshape(-1, W128)
    gi2 = gamma_items.reshape(-1, W128)
    tu2 = theta_users.reshape(-1, W128)
    beta_flat = beta_items.reshape(-1)
    fd, bd = _sc_feat(pi, ni, features, beta_flat)
    e2 = jnp.pad(jnp.concatenate([embedding, visual_bias], axis=1),
                 ((0, 0), (0, W128 - DG - 1)))
    p = _tc_mm(fd, e2)
    ugut, gid128 = _sc_narrow(ui, pi, ni, gu2, gi2, tu2)
    out = _tc_fin(ugut, gid128, p, bd.reshape(NB, 1, TB))
    return out.reshape(B)


# speedup vs baseline: 2.3618x; 1.2708x over previous
import functools

import jax
import jax.numpy as jnp
from jax import lax
from jax.experimental import pallas as pl
from jax.experimental.pallas import tpu as pltpu
from jax.experimental.pallas import tpu_sc as plsc

B = 16384
FEAT = 512
DG = 64
W128 = 128
NW = 32
BPW = B // NW
CF = 32
NCF = BPW // CF
CN = 64
NCN = BPW // CN
NB = 32
TB = B // NB
L = 16
N_ROWS = 100000
TW = 512
NTB = 98
HALF_N = NTB * TW

_mesh = plsc.VectorSubcoreMesh(core_axis_name="c", subcore_axis_name="s")


def _wid_base():
    return (lax.axis_index("s") * 2 + lax.axis_index("c")) * BPW


@functools.partial(
    pl.kernel,
    out_type=(
        jax.ShapeDtypeStruct((B, FEAT), jnp.float32),
        jax.ShapeDtypeStruct((B,), jnp.float32),
    ),
    mesh=_mesh,
    scratch_types=(
        pltpu.VMEM((BPW,), jnp.int32),
        pltpu.VMEM((BPW,), jnp.int32),
        pltpu.VMEM((BPW,), jnp.float32),
        pltpu.VMEM((BPW,), jnp.float32),
        pltpu.VMEM((CF, FEAT), jnp.float32),
        pltpu.VMEM((CF, FEAT), jnp.float32),
        pltpu.VMEM((CF, FEAT), jnp.float32),
        pltpu.VMEM((CF, FEAT), jnp.float32),
        pltpu.SemaphoreType.DMA,
        pltpu.SemaphoreType.DMA,
        pltpu.SemaphoreType.DMA,
        pltpu.SemaphoreType.DMA,
        pltpu.SemaphoreType.DMA,
    ),
)
def _sc_feat(pi_h, ni_h, feat_h, beta_h, fd_o, bd_o,
             pi_v, ni_v, bpv, bnv, fp0, fn0, fp1, fn1,
             semb, semg0, semg1, sems0, sems1):
    base = _wid_base()
    pltpu.sync_copy(pi_h.at[pl.ds(base, BPW)], pi_v)
    pltpu.sync_copy(ni_h.at[pl.ds(base, BPW)], ni_v)

    bcp = pltpu.async_copy(beta_h.at[pi_v], bpv, semb)
    bcn = pltpu.async_copy(beta_h.at[ni_v], bnv, semb)

    fps = (fp0, fp1)
    fns = (fn0, fn1)
    semg = (semg0, semg1)
    sems = (sems0, sems1)

    def fire(ci):
        s = ci % 2
        return [
            pltpu.async_copy(feat_h.at[pi_v.at[pl.ds(ci * CF, CF)]],
                             fps[s], semg[s]),
            pltpu.async_copy(feat_h.at[ni_v.at[pl.ds(ci * CF, CF)]],
                             fns[s], semg[s]),
        ]

    def process(ci, gcp):
        s = ci % 2
        for cp in gcp:
            cp.wait()
        fp, fn = fps[s], fns[s]

        def frow(r, c2):
            for k in range(FEAT // L):
                o = k * L
                fp[r, pl.ds(o, L)] = fp[r, pl.ds(o, L)] - fn[r, pl.ds(o, L)]
            return c2

        lax.fori_loop(0, CF, frow, 0)
        return pltpu.async_copy(fp, fd_o.at[pl.ds(base + ci * CF, CF)],
                                sems[s])

    g = {}
    st = {}
    for ci in range(NCF):
        if ci >= 2:
            st[ci - 2].wait()
        g[ci] = fire(ci)
        if ci >= 1:
            st[ci - 1] = process(ci - 1, g[ci - 1])
    st[NCF - 1] = process(NCF - 1, g[NCF - 1])
    st[NCF - 2].wait()
    st[NCF - 1].wait()

    bcp.wait()
    bcn.wait()

    def bdiff(gi, carry):
        s = gi * L
        bpv[pl.ds(s, L)] = bpv[pl.ds(s, L)] - bnv[pl.ds(s, L)]
        return carry

    lax.fori_loop(0, BPW // L, bdiff, 0, unroll=4)
    pltpu.sync_copy(bpv, bd_o.at[pl.ds(base, BPW)])


@functools.partial(
    pl.kernel,
    out_type=(
        jax.ShapeDtypeStruct((B, W128), jnp.float32),
        jax.ShapeDtypeStruct((B, W128), jnp.float32),
    ),
    mesh=_mesh,
    scratch_types=(
        pltpu.VMEM((BPW + L,), jnp.int32),
        pltpu.VMEM((BPW + L,), jnp.int32),
        pltpu.VMEM((BPW + L,), jnp.int32),
        pltpu.VMEM((BPW,), jnp.int32),
        pltpu.VMEM((BPW,), jnp.int32),
        pltpu.VMEM((BPW,), jnp.int32),
        pltpu.VMEM((BPW,), jnp.int32),
        pltpu.VMEM((BPW,), jnp.int32),
        pltpu.VMEM((BPW,), jnp.int32),
        pltpu.VMEM((CN, W128), jnp.float32),
        pltpu.VMEM((CN, W128), jnp.float32),
        pltpu.VMEM((CN, W128), jnp.float32),
        pltpu.VMEM((CN, W128), jnp.float32),
        pltpu.VMEM((CN, W128), jnp.float32),
        pltpu.VMEM((CN, W128), jnp.float32),
        pltpu.VMEM((CN, W128), jnp.float32),
        pltpu.VMEM((CN, W128), jnp.float32),
        pltpu.SemaphoreType.DMA,
        pltpu.SemaphoreType.DMA,
        pltpu.SemaphoreType.DMA,
        pltpu.SemaphoreType.DMA,
    ),
)
def _sc_narrow(ui_h, pi_h, ni_h, gu_h, gi_h, tu_h, ugut_o, gid_o,
               ui_v, pi_v, ni_v, qu, qp, qn, hu, hp, hn,
               gub0, tub0, gpb0, gnb0, gub1, tub1, gpb1, gnb1,
               semg0, semg1, sems0, sems1):
    base = _wid_base()
    pltpu.sync_copy(ui_h.at[pl.ds(base, BPW)], ui_v.at[pl.ds(0, BPW)])
    pltpu.sync_copy(pi_h.at[pl.ds(base, BPW)], pi_v.at[pl.ds(0, BPW)])
    pltpu.sync_copy(ni_h.at[pl.ds(base, BPW)], ni_v.at[pl.ds(0, BPW)])

    def prep(g, carry):
        s = g * L
        du = ui_v[pl.ds(s, L)] - HALF_N
        dp = pi_v[pl.ds(s, L)] - HALF_N
        dn = ni_v[pl.ds(s, L)] - HALF_N
        mu = lax.shift_right_logical(du, 31)
        mp = lax.shift_right_logical(dp, 31)
        mn = lax.shift_right_logical(dn, 31)
        qu[pl.ds(s, L)] = du + mu * HALF_N
        qp[pl.ds(s, L)] = dp + mp * HALF_N
        qn[pl.ds(s, L)] = dn + mn * HALF_N
        hu[pl.ds(s, L)] = DG - mu * DG
        hp[pl.ds(s, L)] = DG - mp * DG
        hn[pl.ds(s, L)] = DG - mn * DG
        return carry

    lax.fori_loop(0, BPW // L, prep, 0, unroll=4)

    gubs = (gub0, gub1)
    tubs = (tub0, tub1)
    gpbs = (gpb0, gpb1)
    gnbs = (gnb0, gnb1)
    semg = (semg0, semg1)
    sems = (sems0, sems1)

    def fire(ci):
        s = ci % 2
        ix = pl.ds(ci * CN, CN)
        return [
            pltpu.async_copy(gu_h.at[qu.at[ix]], gubs[s], semg[s]),
            pltpu.async_copy(tu_h.at[qu.at[ix]], tubs[s], semg[s]),
            pltpu.async_copy(gi_h.at[qp.at[ix]], gpbs[s], semg[s]),
            pltpu.async_copy(gi_h.at[qn.at[ix]], gnbs[s], semg[s]),
        ]

    def process(ci, gcp):
        s = ci % 2
        for cp in gcp:
            cp.wait()
        gub, tub, gpb, gnb = gubs[s], tubs[s], gpbs[s], gnbs[s]
        off = ci * CN

        def row(r, c2):
            su = hu[pl.ds(off + r, L)][0]
            sp = hp[pl.ds(off + r, L)][0]
            sn = hn[pl.ds(off + r, L)][0]
            for k in range(DG // L):
                o = k * L
                gub[r, pl.ds(o, L)] = gub[r, pl.ds(su + o, L)]
                gub[r, pl.ds(DG + o, L)] = tub[r, pl.ds(su + o, L)]
                gpb[r, pl.ds(o, L)] = (gpb[r, pl.ds(sp + o, L)]
                                       - gnb[r, pl.ds(sn + o, L)])
            return c2

        lax.fori_loop(0, CN, row, 0)
        dst = pl.ds(base + off, CN)
        return [
            pltpu.async_copy(gub, ugut_o.at[dst], sems[s]),
            pltpu.async_copy(gpb, gid_o.at[dst], sems[s]),
        ]

    g = {}
    st = {}
    for ci in range(NCN):
        if ci >= 2:
            for cp in st[ci - 2]:
                cp.wait()
        g[ci] = fire(ci)
        if ci >= 1:
            st[ci - 1] = process(ci - 1, g[ci - 1])
    st[NCN - 1] = process(NCN - 1, g[NCN - 1])
    for ci in (NCN - 2, NCN - 1):
        for cp in st[ci]:
            cp.wait()


def _tc_tr_body(gul_ref, guh_ref, tul_ref, tuh_ref, gil_ref, gih_ref,
                gu_ref, tu_ref, gi_ref):
    gu_ref[...] = jnp.concatenate([gul_ref[...].T, guh_ref[...].T], axis=1)
    tu_ref[...] = jnp.concatenate([tul_ref[...].T, tuh_ref[...].T], axis=1)
    gi_ref[...] = jnp.concatenate([gil_ref[...].T, gih_ref[...].T], axis=1)


_tc_tr = pl.pallas_call(
    _tc_tr_body,
    grid=(NTB,),
    in_specs=[
        pl.BlockSpec((DG, TW), lambda b: (0, b)),
        pl.BlockSpec((DG, TW), lambda b: (0, b + NTB)),
        pl.BlockSpec((DG, TW), lambda b: (0, b)),
        pl.BlockSpec((DG, TW), lambda b: (0, b + NTB)),
        pl.BlockSpec((DG, TW), lambda b: (0, b)),
        pl.BlockSpec((DG, TW), lambda b: (0, b + NTB)),
    ],
    out_specs=[
        pl.BlockSpec((TW, W128), lambda b: (b, 0)),
        pl.BlockSpec((TW, W128), lambda b: (b, 0)),
        pl.BlockSpec((TW, W128), lambda b: (b, 0)),
    ],
    out_shape=[
        jax.ShapeDtypeStruct((HALF_N, W128), jnp.float32),
        jax.ShapeDtypeStruct((HALF_N, W128), jnp.float32),
        jax.ShapeDtypeStruct((HALF_N, W128), jnp.float32),
    ],
)


def _tc_mm_body(fd_ref, e_ref, p_ref):
    p_ref[...] = jnp.dot(fd_ref[...], e_ref[...],
                         preferred_element_type=jnp.float32)


_tc_mm = pl.pallas_call(
    _tc_mm_body,
    grid=(NB,),
    in_specs=[
        pl.BlockSpec((TB, FEAT), lambda b: (b, 0)),
        pl.BlockSpec((FEAT, W128), lambda b: (0, 0)),
    ],
    out_specs=pl.BlockSpec((TB, W128), lambda b: (b, 0)),
    out_shape=jax.ShapeDtypeStruct((B, W128), jnp.float32),
)


def _tc_fin_body(ugut_ref, gid_ref, p_ref, bd_ref, o_ref):
    p = p_ref[...]
    ugut = ugut_ref[...]
    acc = (ugut[:, :DG] * gid_ref[:, :DG]
           + ugut[:, DG:] * p[:, :DG])
    o_ref[...] = (bd_ref[...]
                  + (jnp.sum(acc, axis=1) + p[:, DG])[None, None, :])


_tc_fin = pl.pallas_call(
    _tc_fin_body,
    grid=(NB,),
    in_specs=[
        pl.BlockSpec((TB, W128), lambda b: (b, 0)),
        pl.BlockSpec((TB, W128), lambda b: (b, 0)),
        pl.BlockSpec((TB, W128), lambda b: (b, 0)),
        pl.BlockSpec((1, 1, TB), lambda b: (b, 0, 0)),
    ],
    out_specs=pl.BlockSpec((1, 1, TB), lambda b: (b, 0, 0)),
    out_shape=jax.ShapeDtypeStruct((NB, 1, TB), jnp.float32),
)


def kernel(ui, pi, ni, features, gamma_users, gamma_items, theta_users,
           embedding, beta_items, visual_bias):
    ui = ui.astype(jnp.int32)
    pi = pi.astype(jnp.int32)
    ni = ni.astype(jnp.int32)
    guT = gamma_users.T
    tuT = theta_users.T
    giT = gamma_items.T
    gu2, tu2, gi2 = _tc_tr(guT, guT, tuT, tuT, giT, giT)
    beta_flat = beta_items.reshape(-1)
    fd, bd = _sc_feat(pi, ni, features, beta_flat)
    e2 = jnp.pad(jnp.concatenate([embedding, visual_bias], axis=1),
                 ((0, 0), (0, W128 - DG - 1)))
    p = _tc_mm(fd, e2)
    ugut, gid128 = _sc_narrow(ui, pi, ni, gu2, gi2, tu2)
    out = _tc_fin(ugut, gid128, p, bd.reshape(NB, 1, TB))
    return out.reshape(B)
